# Initial kernel scaffold; baseline (speedup 1.0000x reference)
import jax
import jax.numpy as jnp
from jax.experimental import pallas as pl


def kernel(location_ids, embedding_table):
    raise NotImplementedError("write your pallas kernel here")


# speedup vs baseline: 6.7316x; 6.7316x over previous
import functools

import jax
import jax.numpy as jnp
from jax import lax
from jax.experimental import pallas as pl
from jax.experimental.pallas import tpu as pltpu
from jax.experimental.pallas import tpu_sc as plsc

B = 16384 * 50
D = 64
NC, NS = 2, 16
NW = NC * NS
PT = B // NW
CH = 128
GPC = 4
GROUP = CH * GPC
NG = PT // GROUP
NCH = PT // CH

_mesh = plsc.VectorSubcoreMesh(core_axis_name="c", subcore_axis_name="s")


@functools.partial(
    pl.kernel,
    out_type=jax.ShapeDtypeStruct((B, D), jnp.float32),
    mesh=_mesh,
    scratch_types=[
        pltpu.VMEM((NCH, CH), jnp.int32),
        pltpu.VMEM((GROUP, D), jnp.float32),
        pltpu.SemaphoreType.DMA,
    ],
    compiler_params=pltpu.CompilerParams(use_tc_tiling_on_sc=False),
)
def _gather_kernel(ids_hbm, table_hbm, out_hbm, idx_v, rows_v, sem):
    wid = lax.axis_index("s") * NC + lax.axis_index("c")
    pltpu.sync_copy(ids_hbm.at[pl.ds(wid * NCH, NCH)], idx_v)

    def body(g, carry):
        copies = []
        for u in range(GPC):
            j = g * GPC + u
            copies.append(
                pltpu.async_copy(
                    table_hbm.at[idx_v.at[j]],
                    rows_v.at[pl.ds(u * CH, CH)],
                    sem,
                )
            )
        for c in copies:
            c.wait()
        base = wid * PT + g * GROUP
        pltpu.sync_copy(rows_v, out_hbm.at[pl.ds(base, GROUP)])
        return carry

    lax.fori_loop(0, NG, body, 0)


def kernel(location_ids, embedding_table):
    ids = location_ids.reshape(B // CH, CH)
    out = _gather_kernel(ids, embedding_table)
    return out.reshape(location_ids.shape + (D,))


# speedup vs baseline: 6.9236x; 1.0285x over previous
import functools

import jax
import jax.numpy as jnp
from jax import lax
from jax.experimental import pallas as pl
from jax.experimental.pallas import tpu as pltpu
from jax.experimental.pallas import tpu_sc as plsc

B = 16384 * 50
D = 64
NC, NS = 2, 16
NW = NC * NS
PT = B // NW
CH = 128
GPC = 4
GROUP = CH * GPC
NG = PT // GROUP
NCH = PT // CH

_mesh = plsc.VectorSubcoreMesh(core_axis_name="c", subcore_axis_name="s")


@functools.partial(
    pl.kernel,
    out_type=jax.ShapeDtypeStruct((B, D), jnp.float32),
    mesh=_mesh,
    scratch_types=[
        pltpu.VMEM((NCH, CH), jnp.int32),
        pltpu.VMEM((GROUP, D), jnp.float32),
        pltpu.VMEM((GROUP, D), jnp.float32),
        pltpu.SemaphoreType.DMA,
        pltpu.SemaphoreType.DMA,
        pltpu.SemaphoreType.DMA,
    ],
    compiler_params=pltpu.CompilerParams(use_tc_tiling_on_sc=False),
)
def _gather_kernel(ids_hbm, table_hbm, out_hbm, idx_v, rows0, rows1,
                   sem_g, sem_o0, sem_o1):
    wid = lax.axis_index("s") * NC + lax.axis_index("c")
    pltpu.sync_copy(ids_hbm.at[pl.ds(wid * NCH, NCH)], idx_v)
    rows = (rows0, rows1)
    sem_o = (sem_o0, sem_o1)

    def body(i, carry):
        for p in range(2):
            g = 2 * i + p

            @pl.when(i >= 1)
            def _():
                pltpu.make_async_copy(
                    rows[p], out_hbm.at[pl.ds(0, GROUP)], sem_o[p]
                ).wait()

            copies = []
            for u in range(GPC):
                j = g * GPC + u
                copies.append(
                    pltpu.async_copy(
                        table_hbm.at[idx_v.at[j]],
                        rows[p].at[pl.ds(u * CH, CH)],
                        sem_g,
                    )
                )
            for c in copies:
                c.wait()
            base = wid * PT + g * GROUP
            pltpu.async_copy(rows[p], out_hbm.at[pl.ds(base, GROUP)], sem_o[p])
        return carry

    lax.fori_loop(0, NG // 2, body, 0)
    for p in range(2):
        pltpu.make_async_copy(
            rows[p], out_hbm.at[pl.ds(0, GROUP)], sem_o[p]
        ).wait()


def kernel(location_ids, embedding_table):
    ids = location_ids.reshape(B // CH, CH)
    out = _gather_kernel(ids, embedding_table)
    return out.reshape(location_ids.shape + (D,))


# speedup vs baseline: 7.7292x; 1.1164x over previous
import functools

import jax
import jax.numpy as jnp
from jax import lax
from jax.experimental import pallas as pl
from jax.experimental.pallas import tpu as pltpu
from jax.experimental.pallas import tpu_sc as plsc

B = 16384 * 50
D = 64
NC, NS = 2, 16
NW = NC * NS
PT = B // NW
CH = 128
GPC = 2
GROUP = CH * GPC
NG = PT // GROUP
NCH = PT // CH

_mesh = plsc.VectorSubcoreMesh(core_axis_name="c", subcore_axis_name="s")


@functools.partial(
    pl.kernel,
    out_type=jax.ShapeDtypeStruct((B, D), jnp.float32),
    mesh=_mesh,
    scratch_types=[
        pltpu.VMEM((NCH, CH), jnp.int32),
        pltpu.VMEM((GROUP, D), jnp.float32),
        pltpu.VMEM((GROUP, D), jnp.float32),
        pltpu.VMEM_SHARED((10000, D), jnp.float32),
        pltpu.SemaphoreType.DMA,
        pltpu.SemaphoreType.DMA,
        pltpu.SemaphoreType.DMA,
    ],
    compiler_params=pltpu.CompilerParams(use_tc_tiling_on_sc=False),
)
def _gather_kernel(ids_hbm, table_hbm, out_hbm, idx_v, rows0, rows1,
                   table_sh, sem_g, sem_o0, sem_o1):
    wid = lax.axis_index("s") * NC + lax.axis_index("c")
    sid = lax.axis_index("s")

    tchunk = 10000 // NS
    pltpu.sync_copy(
        table_hbm.at[pl.ds(sid * tchunk, tchunk)],
        table_sh.at[pl.ds(sid * tchunk, tchunk)],
    )
    pltpu.sync_copy(ids_hbm.at[pl.ds(wid * NCH, NCH)], idx_v)
    plsc.subcore_barrier()
    rows = (rows0, rows1)
    sem_o = (sem_o0, sem_o1)

    def body(i, carry):
        for p in range(2):
            g = 2 * i + p

            @pl.when(i >= 1)
            def _():
                pltpu.make_async_copy(
                    rows[p], out_hbm.at[pl.ds(0, GROUP)], sem_o[p]
                ).wait()

            copies = []
            for u in range(GPC):
                j = g * GPC + u
                copies.append(
                    pltpu.async_copy(
                        table_sh.at[idx_v.at[j]],
                        rows[p].at[pl.ds(u * CH, CH)],
                        sem_g,
                    )
                )
            for c in copies:
                c.wait()
            base = wid * PT + g * GROUP
            pltpu.async_copy(rows[p], out_hbm.at[pl.ds(base, GROUP)], sem_o[p])
        return carry

    lax.fori_loop(0, NG // 2, body, 0)
    for p in range(2):
        pltpu.make_async_copy(
            rows[p], out_hbm.at[pl.ds(0, GROUP)], sem_o[p]
        ).wait()


def kernel(location_ids, embedding_table):
    ids = location_ids.reshape(B // CH, CH)
    out = _gather_kernel(ids, embedding_table)
    return out.reshape(location_ids.shape + (D,))


# speedup vs baseline: 7.7345x; 1.0007x over previous
import functools

import jax
import jax.numpy as jnp
from jax import lax
from jax.experimental import pallas as pl
from jax.experimental.pallas import tpu as pltpu
from jax.experimental.pallas import tpu_sc as plsc

B = 16384 * 50
D = 64
V = 10000
NC, NS = 2, 16
NW = NC * NS
PT = B // NW
CH = 128
GPC = 4
GROUP = CH * GPC
NG = PT // GROUP
NCH = PT // CH

_mesh = plsc.VectorSubcoreMesh(core_axis_name="c", subcore_axis_name="s")


@functools.partial(
    pl.kernel,
    out_type=jax.ShapeDtypeStruct((B, D), jnp.float32),
    mesh=_mesh,
    scratch_types=[
        pltpu.VMEM((2, GPC, CH), jnp.int32),
        pltpu.VMEM((GROUP, D), jnp.float32),
        pltpu.VMEM((GROUP, D), jnp.float32),
        pltpu.VMEM_SHARED((V, D), jnp.float32),
        pltpu.SemaphoreType.DMA,
        pltpu.SemaphoreType.DMA,
        pltpu.SemaphoreType.DMA,
        pltpu.SemaphoreType.DMA,
        pltpu.SemaphoreType.DMA,
    ],
    compiler_params=pltpu.CompilerParams(use_tc_tiling_on_sc=False),
)
def _gather_kernel(ids_hbm, table_hbm, out_hbm, idx_v, rows0, rows1,
                   table_sh, sem_g, sem_o0, sem_o1, sem_i0, sem_i1):
    wid = lax.axis_index("s") * NC + lax.axis_index("c")
    sid = lax.axis_index("s")
    rows = (rows0, rows1)
    sem_o = (sem_o0, sem_o1)
    sem_i = (sem_i0, sem_i1)

    tchunk = V // NS
    pltpu.sync_copy(
        table_hbm.at[pl.ds(sid * tchunk, tchunk)],
        table_sh.at[pl.ds(sid * tchunk, tchunk)],
    )
    pltpu.async_copy(ids_hbm.at[pl.ds(wid * NCH, GPC)], idx_v.at[0], sem_i0)
    plsc.subcore_barrier()

    def body(i, carry):
        for p in range(2):
            g = 2 * i + p

            pltpu.make_async_copy(
                ids_hbm.at[pl.ds(0, GPC)], idx_v.at[p], sem_i[p]
            ).wait()

            @pl.when(g + 1 < NG)
            def _():
                pltpu.async_copy(
                    ids_hbm.at[pl.ds(wid * NCH + (g + 1) * GPC, GPC)],
                    idx_v.at[(p + 1) % 2],
                    sem_i[(p + 1) % 2],
                )

            @pl.when(i >= 1)
            def _():
                pltpu.make_async_copy(
                    rows[p], out_hbm.at[pl.ds(0, GROUP)], sem_o[p]
                ).wait()

            copies = []
            for u in range(GPC):
                copies.append(
                    pltpu.async_copy(
                        table_sh.at[idx_v.at[p].at[u]],
                        rows[p].at[pl.ds(u * CH, CH)],
                        sem_g,
                    )
                )
            for c in copies:
                c.wait()
            base = wid * PT + g * GROUP
            pltpu.async_copy(rows[p], out_hbm.at[pl.ds(base, GROUP)], sem_o[p])
        return carry

    lax.fori_loop(0, NG // 2, body, 0)
    for p in range(2):
        pltpu.make_async_copy(
            rows[p], out_hbm.at[pl.ds(0, GROUP)], sem_o[p]
        ).wait()


def kernel(location_ids, embedding_table):
    ids = location_ids.reshape(B // CH, CH)
    out = _gather_kernel(ids, embedding_table)
    return out.reshape(location_ids.shape + (D,))
